# baseline (speedup 1.0000x reference)
import jax
import jax.numpy as jnp
from jax import lax
from jax.experimental import pallas as pl
from jax.experimental.pallas import tpu as pltpu
from jax.experimental.pallas import tpu_sc as plsc

NUM_ATOMS = 100000
NUM_ELEMENTS = 119
EMBED_DIM = 256
BLK = 80
NB = NUM_ATOMS // BLK
NW = 32
BPW = (NB + NW - 1) // NW
L = 16
ROWV = BLK * EMBED_DIM
NSTEP = 13


def _body(idx_hbm, table_hbm, tableflat_hbm, out_hbm, idx_v, table_v,
          cbuf0, cbuf1, gbuf, cwsem0, cwsem1, gwsem, gsem):
    c = lax.axis_index("c")
    s = lax.axis_index("s")
    w = s * 2 + c
    start = 39 * w + jnp.minimum(w, 2)
    nb_w = jnp.where(w < 2, 40, 39)

    pltpu.sync_copy(idx_hbm.at[pl.ds(start * BLK, BPW * BLK)],
                    idx_v.at[pl.ds(0, BPW * BLK)])
    pltpu.sync_copy(tableflat_hbm, table_v)

    cbufs = (cbuf0, cbuf1)
    cwsems = (cwsem0, cwsem1)
    iota16 = lax.iota(jnp.int32, L)

    def wait_write(buf, sem):
        pltpu.make_async_copy(buf, out_hbm.at[pl.ds(0, BLK)], sem).wait()

    def compute_block(b, buf):
        @plsc.parallel_loop(0, BLK, unroll=16)
        def _(r):
            v = idx_v[pl.ds(b * BLK + r, L)]
            base = v[0] * EMBED_DIM
            for cc in range(EMBED_DIM // L):
                buf[r, pl.ds(cc * L, L)] = table_v[pl.ds(base + cc * L, L)]

    def out_slice(b):
        return out_hbm.at[pl.ds((start + b) * BLK, BLK)]

    def step(u, carry):
        bg = 3 * u + 2

        for p in range(2):
            b = 3 * u + p

            @pl.when(b < nb_w)
            def _():
                @pl.when(u >= 1)
                def _():
                    wait_write(cbufs[p], cwsems[p])
                compute_block(b, cbufs[p])
                pltpu.async_copy(cbufs[p], out_slice(b), cwsems[p])

        @pl.when(bg < nb_w)
        def _():
            @pl.when(u >= 1)
            def _():
                wait_write(gbuf, gwsem)
            compute_block(bg, gbuf)
            pltpu.async_copy(gbuf, out_slice(bg), gwsem)

        return carry

    lax.fori_loop(0, NSTEP, step, 0)

    last = 3 * NSTEP

    @pl.when(last < nb_w)
    def _():
        wait_write(cbufs[0], cwsems[0])
        compute_block(last, cbufs[0])
        pltpu.async_copy(cbufs[0], out_slice(last), cwsems[0])

    wait_write(cbufs[0], cwsems[0])
    wait_write(cbufs[1], cwsems[1])
    wait_write(gbuf, gwsem)


def kernel(atomic_numbers, embedding):
    mesh = plsc.VectorSubcoreMesh(core_axis_name="c", subcore_axis_name="s")
    k = pl.kernel(
        _body,
        mesh=mesh,
        compiler_params=pltpu.CompilerParams(needs_layout_passes=False),
        out_type=jax.ShapeDtypeStruct((NUM_ATOMS, EMBED_DIM), jnp.float32),
        scratch_types=[
            pltpu.VMEM((BPW * BLK + L,), jnp.int32),
            pltpu.VMEM((NUM_ELEMENTS * EMBED_DIM,), jnp.float32),
            pltpu.VMEM((BLK, EMBED_DIM), jnp.float32),
            pltpu.VMEM((BLK, EMBED_DIM), jnp.float32),
            pltpu.VMEM((BLK, EMBED_DIM), jnp.float32),
            pltpu.SemaphoreType.DMA,
            pltpu.SemaphoreType.DMA,
            pltpu.SemaphoreType.DMA,
            pltpu.SemaphoreType.DMA,
        ],
    )
    idxflat = atomic_numbers.astype(jnp.int32)
    idxflat = jnp.pad(idxflat, (0, NW * BPW * BLK - NUM_ATOMS))
    return k(idxflat, embedding, embedding.reshape(-1))


# speedup vs baseline: 1.3430x; 1.3430x over previous
import jax
import jax.numpy as jnp
from jax import lax
from jax.experimental import pallas as pl
from jax.experimental.pallas import tpu as pltpu
from jax.experimental.pallas import tpu_sc as plsc

NUM_ATOMS = 100000
NUM_ELEMENTS = 119
EMBED_DIM = 256
BLK = 80
NB = NUM_ATOMS // BLK
NW = 32
BPW = (NB + NW - 1) // NW
L = 16
ROWV = BLK * EMBED_DIM
NSTEP = 13


def _body(idx_hbm, table_hbm, tableflat_hbm, out_hbm, idx_v, table_v,
          cbuf0, cbuf1, gbuf, cwsem0, cwsem1, gwsem, gsem):
    c = lax.axis_index("c")
    s = lax.axis_index("s")
    w = s * 2 + c
    start = 39 * w + jnp.minimum(w, 2)
    nb_w = jnp.where(w < 2, 40, 39)

    pltpu.sync_copy(idx_hbm.at[pl.ds(start * BLK, BPW * BLK)],
                    idx_v.at[pl.ds(0, BPW * BLK)])
    pltpu.sync_copy(tableflat_hbm, table_v)

    cbufs = (cbuf0, cbuf1)
    cwsems = (cwsem0, cwsem1)
    iota16 = lax.iota(jnp.int32, L)

    def wait_write(buf, sem):
        pltpu.make_async_copy(buf, out_hbm.at[pl.ds(0, BLK)], sem).wait()

    def compute_block(b, buf):
        @plsc.parallel_loop(0, BLK, unroll=8)
        def _(r):
            v = idx_v[pl.ds(b * BLK + r, L)]
            base = v[0] * EMBED_DIM
            for cc in range(EMBED_DIM // L):
                buf[r, pl.ds(cc * L, L)] = table_v[pl.ds(base + cc * L, L)]

    def out_slice(b):
        return out_hbm.at[pl.ds((start + b) * BLK, BLK)]

    def step(u, carry):
        bg = 3 * u + 2

        for p in range(2):
            b = 3 * u + p

            @pl.when(b < nb_w)
            def _():
                @pl.when(u >= 1)
                def _():
                    wait_write(cbufs[p], cwsems[p])
                compute_block(b, cbufs[p])
                pltpu.async_copy(cbufs[p], out_slice(b), cwsems[p])

        @pl.when(bg < nb_w)
        def _():
            @pl.when(u >= 1)
            def _():
                wait_write(gbuf, gwsem)
            compute_block(bg, gbuf)
            pltpu.async_copy(gbuf, out_slice(bg), gwsem)

        return carry

    lax.fori_loop(0, NSTEP, step, 0)

    last = 3 * NSTEP

    @pl.when(last < nb_w)
    def _():
        wait_write(cbufs[0], cwsems[0])
        compute_block(last, cbufs[0])
        pltpu.async_copy(cbufs[0], out_slice(last), cwsems[0])

    wait_write(cbufs[0], cwsems[0])
    wait_write(cbufs[1], cwsems[1])
    wait_write(gbuf, gwsem)


def kernel(atomic_numbers, embedding):
    mesh = plsc.VectorSubcoreMesh(core_axis_name="c", subcore_axis_name="s")
    k = pl.kernel(
        _body,
        mesh=mesh,
        compiler_params=pltpu.CompilerParams(needs_layout_passes=False),
        out_type=jax.ShapeDtypeStruct((NUM_ATOMS, EMBED_DIM), jnp.float32),
        scratch_types=[
            pltpu.VMEM((BPW * BLK + L,), jnp.int32),
            pltpu.VMEM((NUM_ELEMENTS * EMBED_DIM,), jnp.float32),
            pltpu.VMEM((BLK, EMBED_DIM), jnp.float32),
            pltpu.VMEM((BLK, EMBED_DIM), jnp.float32),
            pltpu.VMEM((BLK, EMBED_DIM), jnp.float32),
            pltpu.SemaphoreType.DMA,
            pltpu.SemaphoreType.DMA,
            pltpu.SemaphoreType.DMA,
            pltpu.SemaphoreType.DMA,
        ],
    )
    idxflat = atomic_numbers.astype(jnp.int32)
    idxflat = jnp.pad(idxflat, (0, NW * BPW * BLK - NUM_ATOMS))
    return k(idxflat, embedding, embedding.reshape(-1))


# speedup vs baseline: 1.3633x; 1.0151x over previous
import jax
import jax.numpy as jnp
from jax import lax
from jax.experimental import pallas as pl
from jax.experimental.pallas import tpu as pltpu
from jax.experimental.pallas import tpu_sc as plsc

NUM_ATOMS = 100000
NUM_ELEMENTS = 119
EMBED_DIM = 256
BLK = 80
NB = NUM_ATOMS // BLK
NW = 32
BPW = (NB + NW - 1) // NW
L = 16
NSTEP = 13


def _body(idx_hbm, table_hbm, out_hbm, idx_v, table_v,
          buf0, buf1, buf2, wsem0, wsem1, wsem2):
    c = lax.axis_index("c")
    s = lax.axis_index("s")
    w = s * 2 + c
    start = 39 * w + jnp.minimum(w, 2)
    nb_w = jnp.where(w < 2, 40, 39)

    pltpu.sync_copy(idx_hbm.at[pl.ds(start * BLK, BPW * BLK)],
                    idx_v.at[pl.ds(0, BPW * BLK)])
    pltpu.sync_copy(table_hbm, table_v)

    bufs = (buf0, buf1, buf2)
    wsems = (wsem0, wsem1, wsem2)

    def wait_write(p):
        pltpu.make_async_copy(bufs[p], out_hbm.at[pl.ds(0, BLK)],
                              wsems[p]).wait()

    def compute_block(b, buf):
        @plsc.parallel_loop(0, BLK, unroll=8)
        def _(r):
            v = idx_v[pl.ds(b * BLK + r, L)]
            base = v[0] * EMBED_DIM
            for cc in range(EMBED_DIM // L):
                buf[r, pl.ds(cc * L, L)] = table_v[pl.ds(base + cc * L, L)]

    def handle_block(b, p, u):
        @pl.when(b < nb_w)
        def _():
            @pl.when(u >= 1)
            def _():
                wait_write(p)
            compute_block(b, bufs[p])
            pltpu.async_copy(bufs[p],
                             out_hbm.at[pl.ds((start + b) * BLK, BLK)],
                             wsems[p])

    def step(u, carry):
        for p in range(3):
            handle_block(3 * u + p, p, u)
        return carry

    lax.fori_loop(0, NSTEP, step, 0)

    handle_block(3 * NSTEP, 0, NSTEP)

    for p in range(3):
        wait_write(p)


def kernel(atomic_numbers, embedding):
    mesh = plsc.VectorSubcoreMesh(core_axis_name="c", subcore_axis_name="s")
    k = pl.kernel(
        _body,
        mesh=mesh,
        compiler_params=pltpu.CompilerParams(needs_layout_passes=False),
        out_type=jax.ShapeDtypeStruct((NUM_ATOMS, EMBED_DIM), jnp.float32),
        scratch_types=[
            pltpu.VMEM((BPW * BLK + L,), jnp.int32),
            pltpu.VMEM((NUM_ELEMENTS * EMBED_DIM,), jnp.float32),
            pltpu.VMEM((BLK, EMBED_DIM), jnp.float32),
            pltpu.VMEM((BLK, EMBED_DIM), jnp.float32),
            pltpu.VMEM((BLK, EMBED_DIM), jnp.float32),
            pltpu.SemaphoreType.DMA,
            pltpu.SemaphoreType.DMA,
            pltpu.SemaphoreType.DMA,
        ],
    )
    idxflat = atomic_numbers.astype(jnp.int32)
    idxflat = jnp.pad(idxflat, (0, NW * BPW * BLK - NUM_ATOMS))
    return k(idxflat, embedding.reshape(-1))
